# baseline (speedup 1.0000x reference)
import jax
import jax.numpy as jnp
from jax import lax
from jax.experimental import pallas as pl
from jax.experimental.pallas import tpu as pltpu
from jax.experimental.pallas import tpu_sc as plsc

B, N, M = 8, 2048, 2048
L = 16
TR = 16
R = B * N
NT = R // TR
NW = 32
TPW = NT // NW
TILES_PER_BATCH = N // TR
U = 8


def _rsqrt16(ssv):
    iv = lax.bitcast_convert_type(ssv, jnp.int32)
    iv = jnp.int32(0x5F3759DF) - (iv >> 1)
    y = lax.bitcast_convert_type(iv, jnp.float32)
    for _ in range(3):
        y = y * (1.5 - 0.5 * ssv * y * y)
    return y


def _tile_softmax(buf, rows):
    zero8 = [jnp.zeros((L,), jnp.float32)] * U

    def ssq_step(j, accs):
        base = j * U
        out = []
        for u in range(U):
            cols = jnp.full((L,), base + u, jnp.int32)
            v = plsc.load_gather(buf, [rows, cols])
            out.append(accs[u] + v * v)
        return tuple(out)

    accs = lax.fori_loop(0, M // U, ssq_step, tuple(zero8))
    ss = accs[0]
    for u in range(1, U):
        ss = ss + accs[u]
    rv = jnp.minimum(_rsqrt16(ss), jnp.float32(1e12))

    def exp_step(j, saccs):
        base = j * U
        out = []
        for u in range(U):
            cols = jnp.full((L,), base + u, jnp.int32)
            v = plsc.load_gather(buf, [rows, cols])
            e = jnp.exp(v * rv)
            plsc.store_scatter(buf, [rows, cols], e)
            out.append(saccs[u] + e)
        return tuple(out)

    saccs = lax.fori_loop(0, M // U, exp_step, tuple(zero8))
    se = saccs[0]
    for u in range(1, U):
        se = se + saccs[u]
    inv = jnp.ones((L,), jnp.float32) / se

    def scale_step(j, c):
        base = j * U
        for u in range(U):
            cols = jnp.full((L,), base + u, jnp.int32)
            v = plsc.load_gather(buf, [rows, cols])
            plsc.store_scatter(buf, [rows, cols], v * inv)
        return c

    lax.fori_loop(0, M // U, scale_step, jnp.int32(0))


def _zero_rows(buf, lo, hi):
    z = jnp.zeros((L,), jnp.float32)

    def row_step(r, c):
        def col_step(j, c2):
            base = j * (L * U)
            for u in range(U):
                buf[r, pl.ds(base + u * L, L)] = z
            return c2
        return lax.fori_loop(0, M // (L * U), col_step, c)

    lax.fori_loop(lo, hi, row_step, jnp.int32(0))


def _sc_body(s_hbm, nrow_hbm, out_hbm, nrow_v, buf, zbuf):
    wid = lax.axis_index("s") * 2 + lax.axis_index("c")

    pltpu.sync_copy(nrow_hbm, nrow_v)
    nrowf = nrow_v[...].astype(jnp.float32)
    lanes = jnp.arange(L, dtype=jnp.int32)
    _zero_rows(zbuf, 0, TR)

    def tile_step(i, c):
        t = wid + NW * i
        b = t // TILES_PER_BATCH
        start = (t - b * TILES_PER_BATCH) * TR
        nrow_b = jnp.sum(jnp.where(lanes == b, nrowf, 0.0)).astype(jnp.int32)
        nact = jnp.clip(nrow_b - start, 0, TR)

        @pl.when(nact == 0)
        def _():
            pltpu.sync_copy(zbuf, out_hbm.at[pl.ds(t * TR, TR)])

        @pl.when(nact > 0)
        def _():
            pltpu.sync_copy(s_hbm.at[pl.ds(t * TR, TR)], buf)
            _tile_softmax(buf, lanes)
            _zero_rows(buf, nact, TR)
            pltpu.sync_copy(buf, out_hbm.at[pl.ds(t * TR, TR)])

        return c

    lax.fori_loop(0, TPW, tile_step, jnp.int32(0))


def kernel(s, nrow_gt):
    nrow16 = jnp.zeros((L,), jnp.int32).at[:B].set(nrow_gt.astype(jnp.int32))
    s2 = s.reshape(R, M)
    mesh = plsc.VectorSubcoreMesh(core_axis_name="c", subcore_axis_name="s")
    out = pl.kernel(
        _sc_body,
        mesh=mesh,
        compiler_params=pltpu.CompilerParams(needs_layout_passes=False),
        out_type=jax.ShapeDtypeStruct((R, M), jnp.float32),
        scratch_types=[
            pltpu.VMEM((L,), jnp.int32),
            pltpu.VMEM((TR, M), jnp.float32),
            pltpu.VMEM((TR, M), jnp.float32),
        ],
    )(s2, nrow16)
    return out.reshape(B, N, M)


# speedup vs baseline: 7.2295x; 7.2295x over previous
import jax
import jax.numpy as jnp
from jax import lax
from jax.experimental import pallas as pl
from jax.experimental.pallas import tpu as pltpu
from jax.experimental.pallas import tpu_sc as plsc

B, N, M = 8, 2048, 2048
L = 16
TR = 16
R = B * N
NT = R // TR
NW = 32
TPW = NT // NW
TILES_PER_BATCH = N // TR
VPR = M // L
NACC = 8


def _rsqrt16(ssv):
    iv = lax.bitcast_convert_type(ssv, jnp.int32)
    iv = jnp.int32(0x5F3759DF) - (iv >> 1)
    y = lax.bitcast_convert_type(iv, jnp.float32)
    for _ in range(3):
        y = y * (1.5 - 0.5 * ssv * y * y)
    return y


def _row_softmax(buf, r):
    zero = jnp.zeros((L,), jnp.float32)

    accs = [zero] * NACC
    for k in range(VPR):
        v = buf[r, pl.ds(k * L, L)]
        accs[k % NACC] = accs[k % NACC] + v * v
    ss = accs[0]
    for a in accs[1:]:
        ss = ss + a
    ssr = jnp.full((L,), jnp.sum(ss))
    rv = jnp.minimum(_rsqrt16(ssr), jnp.float32(1e12))

    saccs = [zero] * NACC
    for k in range(VPR):
        v = buf[r, pl.ds(k * L, L)]
        e = jnp.exp(v * rv)
        buf[r, pl.ds(k * L, L)] = e
        saccs[k % NACC] = saccs[k % NACC] + e
    se = saccs[0]
    for a in saccs[1:]:
        se = se + a
    inv = jnp.ones((L,), jnp.float32) / jnp.full((L,), jnp.sum(se))

    for k in range(VPR):
        buf[r, pl.ds(k * L, L)] = buf[r, pl.ds(k * L, L)] * inv


def _zero_rows(buf, lo, hi):
    z = jnp.zeros((L,), jnp.float32)

    def row_step(r, c):
        for k in range(VPR):
            buf[r, pl.ds(k * L, L)] = z
        return c

    lax.fori_loop(lo, hi, row_step, jnp.int32(0))


def _sc_body(s_hbm, nrow_hbm, out_hbm, nrow_v, buf, zbuf):
    wid = lax.axis_index("s") * 2 + lax.axis_index("c")

    pltpu.sync_copy(nrow_hbm, nrow_v)
    nrowf = nrow_v[...].astype(jnp.float32)
    lanes = jnp.arange(L, dtype=jnp.int32)
    _zero_rows(zbuf, 0, TR)

    def tile_step(i, c):
        t = wid + NW * i
        b = t // TILES_PER_BATCH
        start = (t - b * TILES_PER_BATCH) * TR
        nrow_b = jnp.sum(jnp.where(lanes == b, nrowf, 0.0)).astype(jnp.int32)
        nact = jnp.clip(nrow_b - start, 0, TR)

        @pl.when(nact == 0)
        def _():
            pltpu.sync_copy(zbuf, out_hbm.at[pl.ds(t * TR, TR)])

        @pl.when(nact > 0)
        def _():
            pltpu.sync_copy(s_hbm.at[pl.ds(t * TR, TR)], buf)

            def row_step(r, c2):
                _row_softmax(buf, r)
                return c2

            lax.fori_loop(0, nact, row_step, jnp.int32(0))
            _zero_rows(buf, nact, TR)
            pltpu.sync_copy(buf, out_hbm.at[pl.ds(t * TR, TR)])

        return c

    lax.fori_loop(0, TPW, tile_step, jnp.int32(0))


def kernel(s, nrow_gt):
    nrow16 = jnp.zeros((L,), jnp.int32).at[:B].set(nrow_gt.astype(jnp.int32))
    s2 = s.reshape(R, M)
    mesh = plsc.VectorSubcoreMesh(core_axis_name="c", subcore_axis_name="s")
    out = pl.kernel(
        _sc_body,
        mesh=mesh,
        compiler_params=pltpu.CompilerParams(needs_layout_passes=False),
        out_type=jax.ShapeDtypeStruct((R, M), jnp.float32),
        scratch_types=[
            pltpu.VMEM((L,), jnp.int32),
            pltpu.VMEM((TR, M), jnp.float32),
            pltpu.VMEM((TR, M), jnp.float32),
        ],
    )(s2, nrow16)
    return out.reshape(B, N, M)
